# baseline (speedup 1.0000x reference)
import jax
import jax.numpy as jnp
from jax.experimental import pallas as pl
from jax.experimental.pallas import tpu as pltpu


_K = 8


def _copy_body(w_hbm, o_hbm, scratch, load_sems, store_sems):
    rows, dim = w_hbm.shape
    r = rows // _K
    loads = [
        pltpu.make_async_copy(
            w_hbm.at[pl.ds(i * r, r), :],
            scratch.at[pl.ds(i * r, r), :],
            load_sems.at[i],
        )
        for i in range(_K)
    ]
    stores = [
        pltpu.make_async_copy(
            scratch.at[pl.ds(i * r, r), :],
            o_hbm.at[0, pl.ds(i * r, r), :],
            store_sems.at[i],
        )
        for i in range(_K)
    ]
    for ld in loads:
        ld.start()
    for i in range(_K):
        loads[i].wait()
        stores[i].start()
    for st in stores:
        st.wait()


def kernel(x, emb_weight):
    seq_len = x.shape[1]
    dim = emb_weight.shape[1]
    out = pl.pallas_call(
        _copy_body,
        out_shape=jax.ShapeDtypeStruct((1, seq_len, dim), emb_weight.dtype),
        in_specs=[pl.BlockSpec(memory_space=pl.ANY)],
        out_specs=pl.BlockSpec(memory_space=pl.ANY),
        scratch_shapes=[
            pltpu.MemorySpace.VMEM((seq_len, dim), emb_weight.dtype),
            pltpu.SemaphoreType.DMA((_K,)),
            pltpu.SemaphoreType.DMA((_K,)),
        ],
    )(emb_weight)
    return out
